# Initial kernel scaffold; baseline (speedup 1.0000x reference)
import jax
import jax.numpy as jnp
from jax.experimental import pallas as pl


def kernel(x, lin_w, lin_b, w1, b1, w2, b2):
    raise NotImplementedError("write your pallas kernel here")


# speedup vs baseline: 126.4212x; 126.4212x over previous
import functools

import jax
import jax.numpy as jnp
from jax.experimental import pallas as pl


def _pick_rows(h):
    for r in (28, 16, 8, 4):
        if h % r == 0:
            return r
    return h


def _fused_gcn_kernel(main_ref, top_ref, bot_ref, w1t_ref, w2t_ref,
                      lin_w_ref, b1_ref, b2_ref, lin_b_ref, out_ref,
                      *, rows, halo, height, width):
    i = pl.program_id(1)
    w = width
    mh = (rows + 2 * halo) * w

    xh = jnp.concatenate([top_ref[0], main_ref[0], bot_ref[0]], axis=1)

    m = jax.lax.broadcasted_iota(jnp.int32, (1, mh), 1)
    row_local = m // w
    col = m - row_local * w
    row = i * rows - halo + row_local
    deg = (5.0
           - (col == 0).astype(jnp.float32)
           - (col == w - 1).astype(jnp.float32)
           - (row == 0).astype(jnp.float32)
           - (row == height - 1).astype(jnp.float32))
    valid = jnp.logical_and(row >= 0, row < height)
    dinv = jnp.where(valid, jax.lax.rsqrt(deg), 0.0)
    mask_l = (col > 0).astype(jnp.float32)
    mask_r = (col < w - 1).astype(jnp.float32)

    def stencil(y):
        f = y.shape[0]
        zw = jnp.zeros((f, w), y.dtype)
        z1 = jnp.zeros((f, 1), y.dtype)
        up = jnp.concatenate([zw, y[:, :-w]], axis=1)
        dn = jnp.concatenate([y[:, w:], zw], axis=1)
        lf = jnp.concatenate([z1, y[:, :-1]], axis=1) * mask_l
        rt = jnp.concatenate([y[:, 1:], z1], axis=1) * mask_r
        return y + up + dn + lf + rt

    prec = jax.lax.Precision.HIGHEST

    z1 = jnp.dot(w1t_ref[...], xh, precision=prec,
                 preferred_element_type=jnp.float32)
    h1 = jax.nn.relu(dinv * stencil(z1 * dinv) + b1_ref[...])
    t = h1 + xh
    z2 = jnp.dot(w2t_ref[...], t, precision=prec,
                 preferred_element_type=jnp.float32)
    h2 = dinv * stencil(z2 * dinv) + b2_ref[...]

    c0 = halo * w
    c1 = c0 + rows * w
    origin = jnp.dot(lin_w_ref[...], xh[:, c0:c1], precision=prec,
                     preferred_element_type=jnp.float32) + lin_b_ref[...]
    out_ref[0] = h2[:, c0:c1] * origin


def kernel(x, lin_w, lin_b, w1, b1, w2, b2):
    bsz, c, h, w = x.shape
    hid = w1.shape[1]
    out_f = w2.shape[1]
    n = h * w
    rows = _pick_rows(h)
    halo = 4
    nblk = h // rows
    hb = halo * w

    x3 = x.reshape(bsz, c, n)
    w1t = w1.T
    w2t = w2.T
    b1c = b1.reshape(hid, 1)
    b2c = b2.reshape(out_f, 1)
    lin_bc = lin_b.reshape(out_f, 1)

    top_blocks = n // hb
    full = lambda a: pl.BlockSpec(a.shape, lambda b, i: (0,) * a.ndim)

    grid_kernel = functools.partial(
        _fused_gcn_kernel, rows=rows, halo=halo, height=h, width=w)

    out3 = pl.pallas_call(
        grid_kernel,
        grid=(bsz, nblk),
        in_specs=[
            pl.BlockSpec((1, c, rows * w), lambda b, i: (b, 0, i)),
            pl.BlockSpec((1, c, hb),
                         lambda b, i: (b, 0, jnp.maximum(i * (rows // halo) - 1, 0))),
            pl.BlockSpec((1, c, hb),
                         lambda b, i: (b, 0, jnp.minimum((i + 1) * (rows // halo),
                                                         top_blocks - 1))),
            full(w1t), full(w2t), full(lin_w), full(b1c), full(b2c),
            full(lin_bc),
        ],
        out_specs=pl.BlockSpec((1, out_f, rows * w), lambda b, i: (b, 0, i)),
        out_shape=jax.ShapeDtypeStruct((bsz, out_f, n), jnp.float32),
    )(x3, x3, x3, w1t, w2t, lin_w, b1c, b2c, lin_bc)

    return out3.reshape(bsz, out_f, h, w)


# speedup vs baseline: 142.5729x; 1.1278x over previous
import functools

import jax
import jax.numpy as jnp
from jax.experimental import pallas as pl


def _pick_rows(h):
    for r in (28, 16, 8, 4):
        if h % r == 0:
            return r
    return h


def _fused_gcn_kernel(main_ref, top_ref, bot_ref, w1t_ref, w2t_ref,
                      lin_w_ref, b1_ref, b2_ref, lin_b_ref, out_ref,
                      *, rows, halo, height, width):
    i = pl.program_id(1)
    w = width
    mh = (rows + 2 * halo) * w

    xh = jnp.concatenate([top_ref[0], main_ref[0], bot_ref[0]], axis=1)

    m = jax.lax.broadcasted_iota(jnp.int32, (1, mh), 1)
    row_local = m // w
    col = m - row_local * w
    row = i * rows - halo + row_local
    deg = (5.0
           - (col == 0).astype(jnp.float32)
           - (col == w - 1).astype(jnp.float32)
           - (row == 0).astype(jnp.float32)
           - (row == height - 1).astype(jnp.float32))
    valid = jnp.logical_and(row >= 0, row < height)
    dinv = jnp.where(valid, jax.lax.rsqrt(deg), 0.0)
    mask_l = (col > 0).astype(jnp.float32)
    mask_r = (col < w - 1).astype(jnp.float32)

    def stencil(y):
        f = y.shape[0]
        zw = jnp.zeros((f, w), y.dtype)
        z1 = jnp.zeros((f, 1), y.dtype)
        up = jnp.concatenate([zw, y[:, :-w]], axis=1)
        dn = jnp.concatenate([y[:, w:], zw], axis=1)
        lf = jnp.concatenate([z1, y[:, :-1]], axis=1) * mask_l
        rt = jnp.concatenate([y[:, 1:], z1], axis=1) * mask_r
        return y + up + dn + lf + rt

    prec = jax.lax.Precision.DEFAULT

    z1 = jnp.dot(w1t_ref[...], xh, precision=prec,
                 preferred_element_type=jnp.float32)
    h1 = jax.nn.relu(dinv * stencil(z1 * dinv) + b1_ref[...])
    t = h1 + xh
    z2 = jnp.dot(w2t_ref[...], t, precision=prec,
                 preferred_element_type=jnp.float32)
    h2 = dinv * stencil(z2 * dinv) + b2_ref[...]

    c0 = halo * w
    c1 = c0 + rows * w
    origin = jnp.dot(lin_w_ref[...], xh[:, c0:c1], precision=prec,
                     preferred_element_type=jnp.float32) + lin_b_ref[...]
    out_ref[0] = h2[:, c0:c1] * origin


def kernel(x, lin_w, lin_b, w1, b1, w2, b2):
    bsz, c, h, w = x.shape
    hid = w1.shape[1]
    out_f = w2.shape[1]
    n = h * w
    rows = _pick_rows(h)
    halo = 4
    nblk = h // rows
    hb = halo * w

    x3 = x.reshape(bsz, c, n)
    w1t = w1.T
    w2t = w2.T
    b1c = b1.reshape(hid, 1)
    b2c = b2.reshape(out_f, 1)
    lin_bc = lin_b.reshape(out_f, 1)

    top_blocks = n // hb
    full = lambda a: pl.BlockSpec(a.shape, lambda b, i: (0,) * a.ndim)

    grid_kernel = functools.partial(
        _fused_gcn_kernel, rows=rows, halo=halo, height=h, width=w)

    out3 = pl.pallas_call(
        grid_kernel,
        grid=(bsz, nblk),
        in_specs=[
            pl.BlockSpec((1, c, rows * w), lambda b, i: (b, 0, i)),
            pl.BlockSpec((1, c, hb),
                         lambda b, i: (b, 0, jnp.maximum(i * (rows // halo) - 1, 0))),
            pl.BlockSpec((1, c, hb),
                         lambda b, i: (b, 0, jnp.minimum((i + 1) * (rows // halo),
                                                         top_blocks - 1))),
            full(w1t), full(w2t), full(lin_w), full(b1c), full(b2c),
            full(lin_bc),
        ],
        out_specs=pl.BlockSpec((1, out_f, rows * w), lambda b, i: (b, 0, i)),
        out_shape=jax.ShapeDtypeStruct((bsz, out_f, n), jnp.float32),
    )(x3, x3, x3, w1t, w2t, lin_w, b1c, b2c, lin_bc)

    return out3.reshape(bsz, out_f, h, w)
